# baseline (device time: 59431 ns/iter reference)
import jax
import jax.numpy as jnp
from jax import lax
from jax.experimental import pallas as pl
from jax.experimental.pallas import tpu as pltpu

N_DEV = 8
CAP = 25
E_PER = 4
BLK = E_PER * CAP


def kernel(x, router_W, route_idx, expert_W):
    del router_W
    n_tok, d_in = x.shape
    _, _, d_out = expert_W.shape
    n_exp = N_DEV * E_PER
    f32 = jnp.float32

    def body(x_ref, idx_ref, w_ref, out_ref, comm_ref, send_sems, recv_sems):
        my = lax.axis_index("i")
        left = lax.rem(my + N_DEV - 1, N_DEV)
        right = lax.rem(my + 1, N_DEV)

        barrier = pltpu.get_barrier_semaphore()
        for nbr in (left, right):
            pl.semaphore_signal(
                barrier, inc=1,
                device_id=(nbr,), device_id_type=pl.DeviceIdType.MESH,
            )
        pl.semaphore_wait(barrier, 2)

        cols_e = lax.broadcasted_iota(jnp.int32, (n_tok, n_exp), 1)
        onehot = (idx_ref[:, :] == cols_e).astype(f32)
        r_i = lax.broadcasted_iota(jnp.int32, (n_tok, n_tok), 0)
        c_i = lax.broadcasted_iota(jnp.int32, (n_tok, n_tok), 1)
        tril = (r_i >= c_i).astype(f32)
        pos = jnp.dot(tril, onehot, preferred_element_type=f32)
        slot = pos * onehot
        slot = slot * (slot <= CAP).astype(f32)

        er = lax.broadcasted_iota(jnp.int32, (n_exp, BLK), 0)
        cr = lax.broadcasted_iota(jnp.int32, (n_exp, BLK), 1)
        sel = (er == my * E_PER + cr // CAP).astype(f32)
        mp = jnp.dot(slot, sel, preferred_element_type=f32)
        kp = lax.broadcasted_iota(jnp.int32, (n_tok, BLK), 1) % CAP + 1
        m = (mp == kp.astype(f32)).astype(f32)
        cx = lax.dot_general(
            m, x_ref[:, :], (((0,), (0,)), ((), ())),
            preferred_element_type=f32,
        )
        for j in range(E_PER):
            comm_ref[0, j * CAP:(j + 1) * CAP, :] = jnp.dot(
                cx[j * CAP:(j + 1) * CAP, :], w_ref[j],
                preferred_element_type=f32,
            )

        def accum(origin, block):
            selo = (er == origin * E_PER + cr // CAP).astype(f32)
            gp = jnp.dot(slot, selo, preferred_element_type=f32)
            g = (gp == kp.astype(f32)).astype(f32)
            return jnp.dot(g, block, preferred_element_type=f32)

        for h in range(N_DEV - 1):
            rdma = pltpu.make_async_remote_copy(
                src_ref=comm_ref.at[h],
                dst_ref=comm_ref.at[h + 1],
                send_sem=send_sems.at[h],
                recv_sem=recv_sems.at[h + 1],
                device_id=(right,),
                device_id_type=pl.DeviceIdType.MESH,
            )
            rdma.start()
            if h == 0:
                out_ref[:, :] = accum(my, comm_ref[0])
            else:
                origin = lax.rem(my + N_DEV - h, N_DEV)
                out_ref[:, :] += accum(origin, comm_ref[h])
            rdma.wait()
        origin = lax.rem(my + 1, N_DEV)
        out_ref[:, :] += accum(origin, comm_ref[N_DEV - 1])

    return pl.pallas_call(
        body,
        out_shape=jax.ShapeDtypeStruct((n_tok, d_out), f32),
        in_specs=[
            pl.BlockSpec(memory_space=pltpu.VMEM),
            pl.BlockSpec(memory_space=pltpu.VMEM),
            pl.BlockSpec(memory_space=pltpu.VMEM),
        ],
        out_specs=pl.BlockSpec(memory_space=pltpu.VMEM),
        scratch_shapes=[
            pltpu.VMEM((N_DEV, BLK, d_out), f32),
            pltpu.SemaphoreType.DMA((N_DEV,)),
            pltpu.SemaphoreType.DMA((N_DEV,)),
        ],
        compiler_params=pltpu.CompilerParams(collective_id=0),
    )(x, route_idx, expert_W)


# device time: 32447 ns/iter; 1.8316x vs baseline; 1.8316x over previous
import functools

import jax
import jax.numpy as jnp
from jax import lax
from jax.experimental import pallas as pl
from jax.experimental.pallas import tpu as pltpu

N_DEV = 8
CAP = 25
E_PER = 4
BLK = E_PER * CAP


def kernel(x, router_W, route_idx, expert_W):
    del router_W
    n_tok, d_in = x.shape
    _, _, d_out = expert_W.shape
    n_exp = N_DEV * E_PER
    f32 = jnp.float32
    bf16 = jnp.bfloat16

    def body(x_ref, idx_ref, w_ref, out_ref, comm_ref, send_sems, recv_sems):
        my = lax.axis_index("i")
        others = [lax.rem(my + k, N_DEV) for k in range(1, N_DEV)]

        barrier = pltpu.get_barrier_semaphore()
        for t in others:
            pl.semaphore_signal(
                barrier, inc=1,
                device_id=(t,), device_id_type=pl.DeviceIdType.MESH,
            )
        pl.semaphore_wait(barrier, N_DEV - 1)

        cols_e = lax.broadcasted_iota(jnp.int32, (n_tok, n_exp), 1)
        onehot = (idx_ref[:, :] == cols_e).astype(f32)
        r_i = lax.broadcasted_iota(jnp.int32, (n_tok, n_tok), 0)
        c_i = lax.broadcasted_iota(jnp.int32, (n_tok, n_tok), 1)
        tril = (r_i >= c_i).astype(f32)
        pos = jnp.dot(tril, onehot, preferred_element_type=f32)
        slot = pos * onehot
        slot = slot * (slot <= CAP).astype(f32)

        er = lax.broadcasted_iota(jnp.int32, (n_exp, BLK), 0)
        cr = lax.broadcasted_iota(jnp.int32, (n_exp, BLK), 1)
        sel = (er == my * E_PER + cr // CAP).astype(f32)
        mp = jnp.dot(slot, sel, preferred_element_type=f32)
        kp = (lax.broadcasted_iota(jnp.int32, (n_tok, BLK), 1) % CAP + 1
              ).astype(f32)
        m = (mp == kp).astype(f32)
        cx = lax.dot_general(
            m, x_ref[:, :], (((0,), (0,)), ((), ())),
            preferred_element_type=f32,
        )
        for j in range(E_PER):
            comm_ref[my, j * CAP:(j + 1) * CAP, :] = jnp.dot(
                cx[j * CAP:(j + 1) * CAP, :], w_ref[j],
                preferred_element_type=f32,
            ).astype(bf16)

        sends = []
        for t in others:
            rdma = pltpu.make_async_remote_copy(
                src_ref=comm_ref.at[my],
                dst_ref=comm_ref.at[my],
                send_sem=send_sems.at[t],
                recv_sem=recv_sems.at[my],
                device_id=(t,),
                device_id_type=pl.DeviceIdType.MESH,
            )
            rdma.start()
            sends.append(rdma)

        er8 = lax.broadcasted_iota(jnp.int32, (n_exp, N_DEV * BLK), 0)
        cr8 = lax.broadcasted_iota(jnp.int32, (n_exp, N_DEV * BLK), 1)
        emat = (er8 == cr8 // CAP).astype(f32)
        aexp = jnp.dot(slot, emat, preferred_element_type=f32)
        kp8 = (lax.broadcasted_iota(jnp.int32, (n_tok, N_DEV * BLK), 1)
               % CAP + 1).astype(f32)
        g = (aexp == kp8).astype(bf16)

        for t in others:
            recv = pltpu.make_async_remote_copy(
                src_ref=comm_ref.at[t],
                dst_ref=comm_ref.at[t],
                send_sem=send_sems.at[t],
                recv_sem=recv_sems.at[t],
                device_id=(t,),
                device_id_type=pl.DeviceIdType.MESH,
            )
            recv.wait_recv()

        gather = jnp.concatenate(
            [comm_ref[s] for s in range(N_DEV)], axis=0
        )
        out_ref[:, :] = jnp.dot(g, gather, preferred_element_type=f32)

        for rdma in sends:
            rdma.wait_send()

        @functools.partial(
            pl.run_scoped, exit_barrier=pltpu.SemaphoreType.REGULAR
        )
        def _(exit_barrier):
            for t in others:
                pl.semaphore_signal(
                    exit_barrier, inc=1,
                    device_id=(t,), device_id_type=pl.DeviceIdType.MESH,
                )
            pl.semaphore_wait(exit_barrier, N_DEV - 1)

    return pl.pallas_call(
        body,
        out_shape=jax.ShapeDtypeStruct((n_tok, d_out), f32),
        in_specs=[
            pl.BlockSpec(memory_space=pltpu.VMEM),
            pl.BlockSpec(memory_space=pltpu.VMEM),
            pl.BlockSpec(memory_space=pltpu.VMEM),
        ],
        out_specs=pl.BlockSpec(memory_space=pltpu.VMEM),
        scratch_shapes=[
            pltpu.VMEM((N_DEV, BLK, d_out), bf16),
            pltpu.SemaphoreType.DMA((N_DEV,)),
            pltpu.SemaphoreType.DMA((N_DEV,)),
        ],
        compiler_params=pltpu.CompilerParams(collective_id=0),
    )(x, route_idx, expert_W)


# device time: 30208 ns/iter; 1.9674x vs baseline; 1.0741x over previous
import functools

import jax
import jax.numpy as jnp
from jax import lax
from jax.experimental import pallas as pl
from jax.experimental.pallas import tpu as pltpu

N_DEV = 8
CAP = 25
E_PER = 4
BLK = E_PER * CAP


def kernel(x, router_W, route_idx, expert_W):
    del router_W
    n_tok, d_in = x.shape
    _, _, d_out = expert_W.shape
    n_exp = N_DEV * E_PER
    f32 = jnp.float32
    bf16 = jnp.bfloat16

    def body(x_ref, idx_ref, w_ref, out_ref, comm_ref, send_sems, recv_sems):
        my = lax.axis_index("i")
        others = [lax.rem(my + k, N_DEV) for k in range(1, N_DEV)]

        barrier = pltpu.get_barrier_semaphore()
        for t in others:
            pl.semaphore_signal(
                barrier, inc=1,
                device_id=(t,), device_id_type=pl.DeviceIdType.MESH,
            )
        pl.semaphore_wait(barrier, N_DEV - 1)

        cols_e = lax.broadcasted_iota(jnp.int32, (n_tok, n_exp), 1)
        onehot = (idx_ref[:, :] == cols_e).astype(bf16)
        r_i = lax.broadcasted_iota(jnp.int32, (n_tok, n_tok), 0)
        c_i = lax.broadcasted_iota(jnp.int32, (n_tok, n_tok), 1)
        tril = (r_i >= c_i).astype(bf16)
        pos = jnp.dot(tril, onehot, preferred_element_type=f32)
        slot = pos * onehot.astype(f32)
        slot = (slot * (slot <= CAP).astype(f32)).astype(bf16)

        er = lax.broadcasted_iota(jnp.int32, (n_exp, BLK), 0)
        cr = lax.broadcasted_iota(jnp.int32, (n_exp, BLK), 1)
        sel = (er == my * E_PER + cr // CAP).astype(bf16)
        mp = jnp.dot(slot, sel, preferred_element_type=f32)
        kp = (lax.broadcasted_iota(jnp.int32, (n_tok, BLK), 1) % CAP + 1
              ).astype(f32)
        m = (mp == kp).astype(bf16)
        cx = lax.dot_general(
            m, x_ref[:, :].astype(bf16), (((0,), (0,)), ((), ())),
            preferred_element_type=f32,
        )
        for j in range(E_PER):
            comm_ref[my, j * CAP:(j + 1) * CAP, :] = jnp.dot(
                cx[j * CAP:(j + 1) * CAP, :], w_ref[j],
                preferred_element_type=f32,
            ).astype(bf16)

        sends = []
        for t in others:
            rdma = pltpu.make_async_remote_copy(
                src_ref=comm_ref.at[my],
                dst_ref=comm_ref.at[my],
                send_sem=send_sems.at[t],
                recv_sem=recv_sems.at[my],
                device_id=(t,),
                device_id_type=pl.DeviceIdType.MESH,
            )
            rdma.start()
            sends.append(rdma)

        er8 = lax.broadcasted_iota(jnp.int32, (n_exp, N_DEV * BLK), 0)
        cr8 = lax.broadcasted_iota(jnp.int32, (n_exp, N_DEV * BLK), 1)
        emat = (er8 == cr8 // CAP).astype(bf16)
        aexp = jnp.dot(slot, emat, preferred_element_type=f32)
        kp8 = (lax.broadcasted_iota(jnp.int32, (n_tok, N_DEV * BLK), 1)
               % CAP + 1).astype(f32)
        g = (aexp == kp8).astype(bf16)

        for t in others:
            recv = pltpu.make_async_remote_copy(
                src_ref=comm_ref.at[t],
                dst_ref=comm_ref.at[t],
                send_sem=send_sems.at[t],
                recv_sem=recv_sems.at[t],
                device_id=(t,),
                device_id_type=pl.DeviceIdType.MESH,
            )
            recv.wait_recv()

        gather = jnp.concatenate(
            [comm_ref[s] for s in range(N_DEV)], axis=0
        )
        out_ref[:, :] = jnp.dot(g, gather, preferred_element_type=f32)

        for rdma in sends:
            rdma.wait_send()

        @functools.partial(
            pl.run_scoped, exit_barrier=pltpu.SemaphoreType.REGULAR
        )
        def _(exit_barrier):
            for t in others:
                pl.semaphore_signal(
                    exit_barrier, inc=1,
                    device_id=(t,), device_id_type=pl.DeviceIdType.MESH,
                )
            pl.semaphore_wait(exit_barrier, N_DEV - 1)

    return pl.pallas_call(
        body,
        out_shape=jax.ShapeDtypeStruct((n_tok, d_out), f32),
        in_specs=[
            pl.BlockSpec(memory_space=pltpu.VMEM),
            pl.BlockSpec(memory_space=pltpu.VMEM),
            pl.BlockSpec(memory_space=pltpu.VMEM),
        ],
        out_specs=pl.BlockSpec(memory_space=pltpu.VMEM),
        scratch_shapes=[
            pltpu.VMEM((N_DEV, BLK, d_out), bf16),
            pltpu.SemaphoreType.DMA((N_DEV,)),
            pltpu.SemaphoreType.DMA((N_DEV,)),
        ],
        compiler_params=pltpu.CompilerParams(collective_id=0),
    )(x, route_idx, expert_W)
